# baseline (speedup 1.0000x reference)
import functools

import jax
import jax.numpy as jnp
from jax import lax
from jax.experimental import pallas as pl
from jax.experimental.pallas import tpu as pltpu
from jax.experimental.pallas import tpu_sc as plsc

GAMMA = 1.0

_DNUMS = lax.GatherDimensionNumbers(offset_dims=(), collapsed_slice_dims=(0,),
                                    start_index_map=(0,))


def _shuf(v, perm):
    return lax.gather(v, perm[:, None], _DNUMS, (1,),
                      mode=lax.GatherScatterMode.PROMISE_IN_BOUNDS)


def _build_sc_kernel(B, K, D, NV, NR):
    info = plsc.get_sparse_core_info()
    NC, NS, L = info.num_cores, info.num_subcores, info.num_lanes
    NW = NC * NS
    assert B % NW == 0
    BPW = B // NW
    BBLK = 16
    NBLK = BPW // BBLK
    NCH = D // L
    REM = D - NCH * L
    NG = K // L

    mesh = plsc.VectorSubcoreMesh(core_axis_name="c", subcore_axis_name="s")

    @functools.partial(
        pl.kernel,
        out_type=jax.ShapeDtypeStruct((B, K), jnp.float32),
        mesh=mesh,
        compiler_params=pltpu.CompilerParams(use_tc_tiling_on_sc=False),
        scratch_types=[
            pltpu.VMEM((BPW,), jnp.int32),
            pltpu.VMEM((BPW,), jnp.int32),
            pltpu.VMEM((BPW, K), jnp.int32),
            pltpu.VMEM((BBLK, D), jnp.float32),
            pltpu.VMEM((BBLK, D), jnp.float32),
            pltpu.VMEM((BBLK, D), jnp.float32),
            pltpu.VMEM((2, K, D), jnp.float32),
            pltpu.VMEM((BBLK, K), jnp.float32),
            pltpu.SemaphoreType.DMA,
            pltpu.SemaphoreType.DMA,
            pltpu.SemaphoreType.DMA,
        ],
    )
    def sc_kernel(hidx_hbm, ridx_hbm, tidx_hbm, ent_hbm, rel_hbm, out_hbm,
                  hidx_v, ridx_v, tidx_v, hrow_v, rrow_v, hr_v, tring_v,
                  oblk_v, sem_a, sem_t0, sem_t1):
        wid = lax.axis_index("s") * NC + lax.axis_index("c")
        b0 = wid * BPW

        pltpu.sync_copy(hidx_hbm.at[pl.ds(b0, BPW)], hidx_v)
        pltpu.sync_copy(ridx_hbm.at[pl.ds(b0, BPW)], ridx_v)
        pltpu.sync_copy(tidx_hbm.at[pl.ds(b0, BPW)], tidx_v)

        lanes = lax.iota(jnp.int32, L)
        tail_mask = lanes >= (L - REM)
        zero = jnp.zeros((L,), jnp.float32)
        bf_perm = [lanes ^ (1 << j) for j in range(4)]
        bf_mask = [(lanes & (1 << j)) == 0 for j in range(4)]

        pltpu.make_async_copy(ent_hbm.at[tidx_v.at[0]],
                              tring_v.at[0], sem_t0).start()
        pltpu.make_async_copy(ent_hbm.at[tidx_v.at[1]],
                              tring_v.at[1], sem_t1).start()

        def process_b(b, row, slot, sem):
            pltpu.make_async_copy(ent_hbm.at[pl.ds(0, K)],
                                  tring_v.at[slot], sem).wait()
            tbuf = tring_v.at[slot]

            hr = [hr_v[row, pl.ds(c * L, L)] for c in range(NCH)]
            hr.append(hr_v[row, pl.ds(D - L, L)])

            def group_body(g, _):
                vs = []
                for kk in range(L):
                    trow = g * L + kk
                    acc = jnp.abs(hr[0] - tbuf[trow, pl.ds(0, L)])
                    for c in range(1, NCH):
                        acc = acc + jnp.abs(hr[c] - tbuf[trow, pl.ds(c * L, L)])
                    d_tail = jnp.abs(hr[NCH] - tbuf[trow, pl.ds(D - L, L)])
                    acc = acc + jnp.where(tail_mask, d_tail, zero)
                    vs.append(acc)
                for j in range(4):
                    perm, m = bf_perm[j], bf_mask[j]
                    vs = [jnp.where(m, a, b)
                          + jnp.where(m, _shuf(a, perm), _shuf(b, perm))
                          for a, b in zip(vs[0::2], vs[1::2])]
                oblk_v[row, pl.ds(g * L, L)] = GAMMA - vs[0]
                return 0

            lax.fori_loop(0, NG, group_body, 0)

            @pl.when(b + 2 < BPW)
            def _():
                pltpu.make_async_copy(ent_hbm.at[tidx_v.at[b + 2]],
                                      tring_v.at[slot], sem).start()

        def blk_body(blk, _):
            bb = blk * BBLK
            cp = pltpu.make_async_copy(ent_hbm.at[hidx_v.at[pl.ds(bb, BBLK)]],
                                       hrow_v, sem_a)
            cp.start()
            cp.wait()
            cp = pltpu.make_async_copy(rel_hbm.at[ridx_v.at[pl.ds(bb, BBLK)]],
                                       rrow_v, sem_a)
            cp.start()
            cp.wait()

            def hr_body(i, _):
                for c in range(NCH):
                    s = pl.ds(c * L, L)
                    hr_v[i, s] = hrow_v[i, s] + rrow_v[i, s]
                s = pl.ds(D - L, L)
                hr_v[i, s] = hrow_v[i, s] + rrow_v[i, s]
                return 0

            lax.fori_loop(0, BBLK, hr_body, 0)

            def pair_body(jj, _):
                b = bb + jj * 2
                process_b(b, jj * 2, 0, sem_t0)
                process_b(b + 1, jj * 2 + 1, 1, sem_t1)
                return 0

            lax.fori_loop(0, BBLK // 2, pair_body, 0)

            pltpu.sync_copy(oblk_v, out_hbm.at[pl.ds(b0 + bb, BBLK)])
            return 0

        lax.fori_loop(0, NBLK, blk_body, 0)

    return sc_kernel


def kernel(head_index, relation_index, tail_index, entity_embedding,
           relation_embedding):
    B, K = tail_index.shape
    NV, D = entity_embedding.shape
    NR = relation_embedding.shape[0]
    fn = _build_sc_kernel(B, K, D, NV, NR)
    return fn(head_index.astype(jnp.int32),
              relation_index.astype(jnp.int32),
              tail_index.astype(jnp.int32),
              entity_embedding, relation_embedding)
